# baseline (speedup 1.0000x reference)
import functools

import jax
import jax.numpy as jnp
import numpy as np
from jax import lax
from jax.experimental import pallas as pl
from jax.experimental.pallas import tpu as pltpu
from jax.experimental.pallas import tpu_sc as plsc

_VOCAB = 100000
_EMBED = 128
_BATCH = 1024
_SEQ = 200

_NW = 32
_ROWS = _BATCH * _SEQ
_CH = 64
_NBLK = _BATCH // _CH
_NCH = _SEQ * _NBLK // _NW
_NBUF = 8
_UNROLL = 2

_OUT_IDX = np.arange(_BATCH)[:, None] * _SEQ + np.arange(_SEQ)[None, :]
_OUT_IDX = np.ascontiguousarray(
    _OUT_IDX.T.reshape(_SEQ, _NBLK, _CH).reshape(_NW, _NCH, _CH)
).astype(np.int32)

_mesh = plsc.VectorSubcoreMesh(core_axis_name="c", subcore_axis_name="s")


@functools.partial(
    pl.kernel,
    mesh=_mesh,
    out_type=jax.ShapeDtypeStruct((_ROWS, _EMBED), jnp.float32),
    scratch_types=[
        pltpu.VMEM((_NCH, _CH), jnp.int32),
        pltpu.VMEM((_NCH, _CH), jnp.int32),
        pltpu.VMEM((_SEQ, _EMBED), jnp.float32),
    ]
    + [pltpu.VMEM((_CH, _EMBED), jnp.float32)] * _NBUF
    + [pltpu.SemaphoreType.DMA] * _NBUF
    + [pltpu.SemaphoreType.DMA] * _NBUF
    + [pltpu.SemaphoreType.DMA],
)
def _emb_lookup(x_hbm, wt_hbm, pt_hbm, oidx_hbm, out_hbm, idx_v, oidx_v,
                pos_v, *bufs_and_sems):
    bufs = bufs_and_sems[:_NBUF]
    gsems = bufs_and_sems[_NBUF:2 * _NBUF]
    wsems = bufs_and_sems[2 * _NBUF:3 * _NBUF]
    ssem = bufs_and_sems[3 * _NBUF]
    wid = lax.axis_index("s") * 2 + lax.axis_index("c")

    pltpu.sync_copy(x_hbm.at[wid], idx_v)

    def gather(c, buf, sem):
        pltpu.async_copy(wt_hbm.at[idx_v.at[c]], buf, sem)

    def gather_wait(buf, sem):
        pltpu.make_async_copy(wt_hbm.at[pl.ds(0, _CH)], buf, sem).wait()

    def scatter(c, buf, sem):
        pltpu.async_copy(buf, out_hbm.at[oidx_v.at[c]], sem)

    def scatter_wait(c, buf, sem):
        pltpu.make_async_copy(buf, out_hbm.at[oidx_v.at[c]], sem).wait()

    def add_pos(c, buf):
        s_pos = (wid * _NCH + c) // _NBLK
        prow = [pos_v[s_pos, pl.ds(j * 16, 16)] for j in range(_EMBED // 16)]

        def row_body(i, carry):
            r = i * _UNROLL
            for u in range(_UNROLL):
                w = [buf[r + u, pl.ds(j * 16, 16)] for j in range(_EMBED // 16)]
                for j in range(_EMBED // 16):
                    buf[r + u, pl.ds(j * 16, 16)] = w[j] + prow[j]
            return carry

        lax.fori_loop(0, _CH // _UNROLL, row_body, 0)

    _HALF = _NBUF // 2

    def chunk_step(c, b, issue_next):
        gather_wait(bufs[b], gsems[b])
        add_pos(c, bufs[b])
        scatter(c, bufs[b], wsems[b])
        b4 = (b + _HALF) % _NBUF
        if issue_next:
            @pl.when(c >= _HALF)
            def _():
                scatter_wait(c - _HALF, bufs[b4], wsems[b4])

            gather(c + _HALF, bufs[b4], gsems[b4])
        else:
            scatter_wait(c - _HALF, bufs[b4], wsems[b4])

    for k in range(_HALF):
        gather(k, bufs[k], gsems[k])
    pltpu.async_copy(oidx_hbm.at[wid], oidx_v, ssem)
    pltpu.async_copy(pt_hbm.at[pl.ds(0, _SEQ)], pos_v, ssem)
    pltpu.make_async_copy(oidx_hbm.at[wid], oidx_v, ssem).wait()
    pltpu.make_async_copy(pt_hbm.at[pl.ds(0, _SEQ)], pos_v, ssem).wait()

    def ring_body(q, carry):
        c0 = _NBUF * q
        for u in range(_NBUF):
            chunk_step(c0 + u, u, issue_next=True)
        return carry

    _NRING = (_NCH - _HALF) // _NBUF
    lax.fori_loop(0, _NRING, ring_body, 0)

    _TAIL = _NCH - _NRING * _NBUF
    for k in range(_TAIL):
        c = _NCH - _TAIL + k
        chunk_step(c, c % _NBUF, issue_next=(k + _HALF < _TAIL))
    for k in range(_HALF):
        c = _NCH - _HALF + k
        scatter_wait(c, bufs[c % _NBUF], wsems[c % _NBUF])


def kernel(x, word_table, pos_table):
    xt = x.astype(jnp.int32).T.reshape(_SEQ, _NBLK, _CH).reshape(_NW, _NCH, _CH)
    out = _emb_lookup(xt, word_table, pos_table, jnp.asarray(_OUT_IDX))
    return out.reshape(_BATCH, _SEQ, _EMBED)


# speedup vs baseline: 1.0028x; 1.0028x over previous
import functools

import jax
import jax.numpy as jnp
import numpy as np
from jax import lax
from jax.experimental import pallas as pl
from jax.experimental.pallas import tpu as pltpu
from jax.experimental.pallas import tpu_sc as plsc

_VOCAB = 100000
_EMBED = 128
_BATCH = 1024
_SEQ = 200

_NW = 32
_ROWS = _BATCH * _SEQ
_CH = 64
_NBLK = _BATCH // _CH
_NCH = _SEQ * _NBLK // _NW
_NBUF = 8
_UNROLL = 2

_OUT_IDX = np.arange(_BATCH)[:, None] * _SEQ + np.arange(_SEQ)[None, :]
_OUT_IDX = np.ascontiguousarray(
    _OUT_IDX.T.reshape(_SEQ, _NBLK, _CH).reshape(_NW, _NCH, _CH)
).astype(np.int32)

_mesh = plsc.VectorSubcoreMesh(core_axis_name="c", subcore_axis_name="s")


@functools.partial(
    pl.kernel,
    mesh=_mesh,
    out_type=jax.ShapeDtypeStruct((_ROWS, _EMBED), jnp.float32),
    scratch_types=[
        pltpu.VMEM((_NCH, _CH), jnp.int32),
        pltpu.VMEM((_NCH, _CH), jnp.int32),
        pltpu.VMEM((_SEQ, _EMBED), jnp.float32),
    ]
    + [pltpu.VMEM((_CH, _EMBED), jnp.float32)] * _NBUF
    + [pltpu.SemaphoreType.DMA] * _NBUF
    + [pltpu.SemaphoreType.DMA] * _NBUF
    + [pltpu.SemaphoreType.DMA],
)
def _emb_lookup(x_hbm, wt_hbm, pt_hbm, oidx_hbm, out_hbm, idx_v, oidx_v,
                pos_v, *bufs_and_sems):
    bufs = bufs_and_sems[:_NBUF]
    gsems = bufs_and_sems[_NBUF:2 * _NBUF]
    wsems = bufs_and_sems[2 * _NBUF:3 * _NBUF]
    ssem = bufs_and_sems[3 * _NBUF]
    wid = lax.axis_index("s") * 2 + lax.axis_index("c")

    pltpu.sync_copy(x_hbm.at[wid], idx_v)

    def gather(c, buf, sem):
        pltpu.async_copy(wt_hbm.at[idx_v.at[c]], buf, sem)

    def gather_wait(buf, sem):
        pltpu.make_async_copy(wt_hbm.at[pl.ds(0, _CH)], buf, sem).wait()

    def scatter(c, buf, sem):
        pltpu.async_copy(buf, out_hbm.at[oidx_v.at[c]], sem)

    def scatter_wait(c, buf, sem):
        pltpu.make_async_copy(buf, out_hbm.at[oidx_v.at[c]], sem).wait()

    def add_pos(c, buf):
        s_pos = (wid * _NCH + c) // _NBLK
        prow = [pos_v[s_pos, pl.ds(j * 16, 16)] for j in range(_EMBED // 16)]

        def row_body(i, carry):
            r = i * _UNROLL
            for u in range(_UNROLL):
                w = [buf[r + u, pl.ds(j * 16, 16)] for j in range(_EMBED // 16)]
                for j in range(_EMBED // 16):
                    buf[r + u, pl.ds(j * 16, 16)] = w[j] + prow[j]
            return carry

        lax.fori_loop(0, _CH // _UNROLL, row_body, 0)

    _HALF = _NBUF // 2

    def chunk_step(c, b, issue_next):
        gather_wait(bufs[b], gsems[b])
        add_pos(c, bufs[b])
        b4 = (b + _HALF) % _NBUF
        if issue_next:
            @pl.when(c >= _HALF)
            def _():
                scatter_wait(c - _HALF, bufs[b4], wsems[b4])

            gather(c + _HALF, bufs[b4], gsems[b4])
        else:
            scatter_wait(c - _HALF, bufs[b4], wsems[b4])
        scatter(c, bufs[b], wsems[b])

    for k in range(_HALF):
        gather(k, bufs[k], gsems[k])
    pltpu.async_copy(oidx_hbm.at[wid], oidx_v, ssem)
    pltpu.async_copy(pt_hbm.at[pl.ds(0, _SEQ)], pos_v, ssem)
    pltpu.make_async_copy(oidx_hbm.at[wid], oidx_v, ssem).wait()
    pltpu.make_async_copy(pt_hbm.at[pl.ds(0, _SEQ)], pos_v, ssem).wait()

    def ring_body(q, carry):
        c0 = _NBUF * q
        for u in range(_NBUF):
            chunk_step(c0 + u, u, issue_next=True)
        return carry

    _NRING = (_NCH - _HALF) // _NBUF
    lax.fori_loop(0, _NRING, ring_body, 0)

    _TAIL = _NCH - _NRING * _NBUF
    for k in range(_TAIL):
        c = _NCH - _TAIL + k
        chunk_step(c, c % _NBUF, issue_next=(k + _HALF < _TAIL))
    for k in range(_HALF):
        c = _NCH - _HALF + k
        scatter_wait(c, bufs[c % _NBUF], wsems[c % _NBUF])


def kernel(x, word_table, pos_table):
    xt = x.astype(jnp.int32).T.reshape(_SEQ, _NBLK, _CH).reshape(_NW, _NCH, _CH)
    out = _emb_lookup(xt, word_table, pos_table, jnp.asarray(_OUT_IDX))
    return out.reshape(_BATCH, _SEQ, _EMBED)
